# Initial kernel scaffold; baseline (speedup 1.0000x reference)
import jax
import jax.numpy as jnp
from jax.experimental import pallas as pl


def kernel(x, y):
    raise NotImplementedError("write your pallas kernel here")


# speedup vs baseline: 55.2283x; 55.2283x over previous
import functools

import jax
import jax.numpy as jnp
from jax import lax
from jax.experimental import pallas as pl
from jax.experimental.pallas import tpu as pltpu
from jax.experimental.pallas import tpu_sc as plsc

ROWS = 128
COLS = 100000
LANES = 16
NWORKERS = 32
RPW = ROWS // NWORKERS
NVEC = COLS // LANES

_NEG = -3.0e38


def _sc_row_stats(x, yb):
    mesh = plsc.VectorSubcoreMesh(core_axis_name="c", subcore_axis_name="s")

    @functools.partial(
        pl.kernel,
        out_type=jax.ShapeDtypeStruct((ROWS, LANES), jnp.float32),
        mesh=mesh,
        scratch_types=[
            pltpu.VMEM((COLS,), jnp.float32),
            pltpu.VMEM((LANES,), jnp.int32),
            pltpu.VMEM((RPW, LANES), jnp.float32),
        ],
    )
    def k(x_hbm, y_hbm, out_hbm, rowbuf, ybuf, obuf):
        wid = lax.axis_index("s") * 2 + lax.axis_index("c")
        base = wid * RPW
        iota = lax.iota(jnp.int32, LANES)
        for r in range(RPW):
            row = base + r
            pltpu.sync_copy(x_hbm.at[row], rowbuf)
            pltpu.sync_copy(y_hbm.at[row], ybuf)
            ys = ybuf[...][0]
            lane = ys & (LANES - 1)
            grp = rowbuf[pl.ds(ys - lane, LANES)]
            fyv = grp[jnp.broadcast_to(lane, (LANES,))]

            def body(i, c):
                t_sum, t1, t2, t3, t4, t5 = c
                v = rowbuf[pl.ds(i * LANES, LANES)]
                t_sum = t_sum + jnp.exp(v)
                m = jnp.minimum(t1, v)
                t1 = jnp.maximum(t1, v)
                v = m
                m = jnp.minimum(t2, v)
                t2 = jnp.maximum(t2, v)
                v = m
                m = jnp.minimum(t3, v)
                t3 = jnp.maximum(t3, v)
                v = m
                m = jnp.minimum(t4, v)
                t4 = jnp.maximum(t4, v)
                v = m
                t5 = jnp.maximum(t5, v)
                return (t_sum, t1, t2, t3, t4, t5)

            z = jnp.zeros((LANES,), jnp.float32)
            neg = jnp.full((LANES,), _NEG, jnp.float32)
            t_sum, t1, t2, t3, t4, t5 = lax.fori_loop(
                0, NVEC, body, (z, neg, neg, neg, neg, neg)
            )

            gvec = jnp.zeros((LANES,), jnp.float32)
            g = jnp.zeros((LANES,), jnp.float32)
            for kk in range(5):
                g = t1
                for sh in (1, 2, 4, 8):
                    g = jnp.maximum(g, g[iota ^ sh])
                cand = jnp.where(t1 == g, iota, LANES)
                for sh in (1, 2, 4, 8):
                    cand = jnp.minimum(cand, cand[iota ^ sh])
                pm = iota == cand
                gvec = jnp.where(iota == kk, g, gvec)
                t1 = jnp.where(pm, t2, t1)
                t2 = jnp.where(pm, t3, t2)
                t3 = jnp.where(pm, t4, t3)
                t4 = jnp.where(pm, t5, t4)
                t5 = jnp.where(pm, _NEG, t5)

            for sh in (1, 2, 4, 8):
                t_sum = t_sum + t_sum[iota ^ sh]
            t_top5 = jnp.where(iota < 5, jnp.exp(gvec), 0.0)
            for sh in (1, 2, 4, 8):
                t_top5 = t_top5 + t_top5[iota ^ sh]
            sv = (t_sum - t_top5) / jnp.exp(fyv) - jnp.where(
                fyv >= g, 0.0, 1.0
            )
            obuf[r, :] = sv
        pltpu.sync_copy(obuf, out_hbm.at[pl.ds(base, RPW)])

    return k(x, yb)


def _tc_finish(s):
    def body(s_ref, o_ref):
        col = s_ref[:, 0:1]
        tot = jnp.sum(jnp.log(1.0 + col), axis=0, keepdims=True)
        o_ref[...] = tot * (1.0 / ROWS)

    return pl.pallas_call(
        body,
        out_shape=jax.ShapeDtypeStruct((1, 1), jnp.float32),
    )(s)


@jax.jit
def kernel(x, y):
    yb = jnp.broadcast_to(y.astype(jnp.int32)[:, None], (ROWS, LANES))
    s = _sc_row_stats(x, yb)
    return _tc_finish(s)[0, 0]
